# Initial kernel scaffold; baseline (speedup 1.0000x reference)
import jax
import jax.numpy as jnp
from jax.experimental import pallas as pl


def kernel(inputs, tables, W, b):
    raise NotImplementedError("write your pallas kernel here")


# speedup vs baseline: 7.3521x; 7.3521x over previous
import functools

import jax
import jax.numpy as jnp
from jax import lax
from jax.experimental import pallas as pl
from jax.experimental.pallas import tpu as pltpu
from jax.experimental.pallas import tpu_sc as plsc

N_TABLES = 26
DIM = 32
NC, NS = 2, 16
NW = NC * NS


def _gather_body(idx_hbm, table_hbm, out_hbm, idx_v, rows_v, sem):
    wid = lax.axis_index("s") * NC + lax.axis_index("c")
    b_per_w = idx_hbm.shape[1] // NW
    base = wid * b_per_w
    pltpu.sync_copy(idx_hbm.at[:, pl.ds(base, b_per_w)], idx_v)

    @pl.loop(0, N_TABLES)
    def _table_loop(j):
        pltpu.async_copy(table_hbm.at[idx_v.at[j]], rows_v, sem).wait()
        pltpu.sync_copy(rows_v, out_hbm.at[j, pl.ds(base, b_per_w), :])


def _sc_gather(flat_idx, flat_table):
    b = flat_idx.shape[1]
    b_per_w = b // NW
    mesh = plsc.VectorSubcoreMesh(core_axis_name="c", subcore_axis_name="s")
    return pl.kernel(
        _gather_body,
        out_type=jax.ShapeDtypeStruct((N_TABLES, b, DIM), jnp.float32),
        mesh=mesh,
        scratch_types=[
            pltpu.VMEM((N_TABLES, b_per_w), jnp.int32),
            pltpu.VMEM((b_per_w, DIM), jnp.float32),
            pltpu.SemaphoreType.DMA,
        ],
        compiler_params=pltpu.CompilerParams(use_tc_tiling_on_sc=False),
    )(flat_idx, flat_table)


def _mm_body(emb_ref, w_ref, b_ref, out_ref):
    acc = b_ref[...]
    for j in range(N_TABLES):
        acc = acc + jnp.dot(
            emb_ref[j], w_ref[j * DIM : (j + 1) * DIM, :],
            preferred_element_type=jnp.float32,
        )
    out_ref[...] = acc


def _tc_matmul(emb, w, b2d):
    _, batch, dim = emb.shape
    n_out = w.shape[1]
    blk = 2048
    return pl.pallas_call(
        _mm_body,
        grid=(batch // blk,),
        in_specs=[
            pl.BlockSpec((N_TABLES, blk, dim), lambda i: (0, i, 0)),
            pl.BlockSpec(w.shape, lambda i: (0, 0)),
            pl.BlockSpec((1, n_out), lambda i: (0, 0)),
        ],
        out_specs=pl.BlockSpec((blk, n_out), lambda i: (i, 0)),
        out_shape=jax.ShapeDtypeStruct((batch, n_out), jnp.float32),
    )(emb, w, b2d)


@jax.jit
def kernel(inputs, tables, W, b):
    n, vocab, dim = tables.shape
    flat_table = tables.reshape(n * vocab, dim)
    offs = (jnp.arange(n, dtype=jnp.int32) * vocab)[:, None]
    flat_idx = inputs + offs
    emb = _sc_gather(flat_idx, flat_table)
    return _tc_matmul(emb, W, b.reshape(1, -1))
